# baseline (speedup 1.0000x reference)
import functools
import math

import jax
import jax.numpy as jnp
from jax.experimental import pallas as pl
from jax.experimental.pallas import tpu as pltpu


def _g_body(oh_ref, wc_ref, g_ref):
    g_ref[...] = jax.lax.dot_general(
        oh_ref[...], wc_ref[...],
        dimension_numbers=(((1,), (1,)), ((), ())),
        preferred_element_type=jnp.float32)


def _pool_body(f_ref, g_ref, pooled_ref, *, keep_cnt, inv_hw):
    f = f_ref[...]
    g = g_ref[...]
    sm = jnp.sum(f * g[:, :, None], axis=1)
    cmp = (sm[:, :, None] >= sm[:, None, :]).astype(jnp.float32)
    cnt = jnp.sum(cmp, axis=1)
    mask = (cnt >= keep_cnt).astype(jnp.float32)
    pooled_ref[...] = jnp.sum(f * mask[:, None, :], axis=2) * inv_hw


def _softmax(x):
    m = jnp.max(x, axis=1, keepdims=True)
    e = jnp.exp(x - m)
    return e / jnp.sum(e, axis=1, keepdims=True)


def _change_body(pooled_ref, wc_ref, bc_ref, preds_ref, oh_ref,
                 logits_ref, change_ref, *, eps):
    pooled = pooled_ref[...]
    logits_after = jax.lax.dot_general(
        pooled, wc_ref[...],
        dimension_numbers=(((1,), (0,)), ((), ())),
        preferred_element_type=jnp.float32) + bc_ref[...]
    preds = preds_ref[...]
    oh = oh_ref[...]
    before = jnp.sum(oh * _softmax(preds), axis=1, keepdims=True)
    after = jnp.sum(oh * _softmax(logits_after), axis=1, keepdims=True)
    logits_ref[...] = logits_after
    change_ref[...] = jnp.maximum(before - after - eps, 0.0)


def _loss_body(change_ref, change_row_ref, logits_ref, preds_ref, oh_ref,
               out_ref, *, n_rows, th_idx):
    change = change_ref[...]
    change_row = change_row_ref[...]
    cnt_lt = jnp.sum((change_row < change).astype(jnp.float32),
                     axis=1, keepdims=True)
    drop = (cnt_lt >= float(n_rows - th_idx)).astype(jnp.float32)
    logits = drop * logits_ref[...] + (1.0 - drop) * preds_ref[...]
    m = jnp.max(logits, axis=1, keepdims=True)
    lse = jnp.log(jnp.sum(jnp.exp(logits - m), axis=1, keepdims=True)) + m
    out_ref[...] = -jnp.sum(oh_ref[...] * (logits - lse), axis=(0, 1),
                            keepdims=True) / float(n_rows)


@jax.jit
def kernel(features, preds, labels, oh_labels, Wc, bc):
    N, C, H, W = features.shape
    HW = H * W
    K = preds.shape[1]
    f = features.reshape(N, C, HW)

    Kp = ((K + 127) // 128) * 128
    oh_p = jnp.pad(oh_labels, ((0, 0), (0, Kp - K)))
    wc_p = jnp.pad(Wc, ((0, 0), (0, Kp - K)))
    G = pl.pallas_call(
        _g_body,
        out_shape=jax.ShapeDtypeStruct((N, C), jnp.float32),
    )(oh_p, wc_p)

    B = 16
    keep_cnt = float(math.ceil(HW / 3.0) + 1)
    pooled = pl.pallas_call(
        functools.partial(_pool_body, keep_cnt=keep_cnt, inv_hw=1.0 / HW),
        grid=(N // B,),
        in_specs=[
            pl.BlockSpec((B, C, HW), lambda i: (i, 0, 0)),
            pl.BlockSpec((B, C), lambda i: (i, 0)),
        ],
        out_specs=pl.BlockSpec((B, C), lambda i: (i, 0)),
        out_shape=jax.ShapeDtypeStruct((N, C), jnp.float32),
    )(f, G)

    logits_after, change = pl.pallas_call(
        functools.partial(_change_body, eps=1e-4),
        out_shape=(jax.ShapeDtypeStruct((N, K), jnp.float32),
                   jax.ShapeDtypeStruct((N, 1), jnp.float32)),
    )(pooled, Wc, bc.reshape(1, K), preds, oh_labels)

    th_idx = int(round(float(N) * 0.3333))
    loss = pl.pallas_call(
        functools.partial(_loss_body, n_rows=N, th_idx=th_idx),
        out_shape=jax.ShapeDtypeStruct((1, 1), jnp.float32),
    )(change, change.reshape(1, N), logits_after, preds, oh_labels)
    return loss[0, 0]


# speedup vs baseline: 5.6637x; 5.6637x over previous
import functools
import math

import jax
import jax.numpy as jnp
from jax.experimental import pallas as pl
from jax.experimental.pallas import tpu as pltpu


def _g_body(oh_ref, wc_ref, g_ref):
    g_ref[...] = jax.lax.dot_general(
        oh_ref[...], wc_ref[...],
        dimension_numbers=(((1,), (1,)), ((), ())),
        preferred_element_type=jnp.float32)


def _pool_body(f_ref, g_ref, pooled_ref, *, keep_cnt, inv_hw):
    ft = f_ref[...]
    g = g_ref[...]
    sm = jnp.sum(ft * g[None, :, :], axis=2)
    smt = jnp.transpose(sm)
    hw = sm.shape[0]
    cnt = jnp.zeros_like(smt)
    for j in range(hw):
        col_j = jnp.broadcast_to(smt[:, j:j + 1], smt.shape)
        cnt += (col_j >= smt).astype(jnp.float32)
    mask = jnp.transpose((cnt >= keep_cnt).astype(jnp.float32))
    pooled_ref[...] = jnp.sum(ft * mask[:, :, None], axis=0) * inv_hw


def _softmax(x):
    m = jnp.max(x, axis=1, keepdims=True)
    e = jnp.exp(x - m)
    return e / jnp.sum(e, axis=1, keepdims=True)


def _change_body(pooled_ref, wc_ref, bc_ref, preds_ref, oh_ref,
                 logits_ref, change_ref, *, eps):
    pooled = pooled_ref[...]
    logits_after = jax.lax.dot_general(
        pooled, wc_ref[...],
        dimension_numbers=(((1,), (0,)), ((), ())),
        preferred_element_type=jnp.float32) + bc_ref[...]
    preds = preds_ref[...]
    oh = oh_ref[...]
    before = jnp.sum(oh * _softmax(preds), axis=1, keepdims=True)
    after = jnp.sum(oh * _softmax(logits_after), axis=1, keepdims=True)
    logits_ref[...] = logits_after
    change_ref[...] = jnp.maximum(before - after - eps, 0.0)


def _loss_body(change_ref, change_row_ref, logits_ref, preds_ref, oh_ref,
               out_ref, *, n_rows, th_idx):
    change = change_ref[...]
    change_row = change_row_ref[...]
    cnt_lt = jnp.sum((change_row < change).astype(jnp.float32),
                     axis=1, keepdims=True)
    drop = (cnt_lt >= float(n_rows - th_idx)).astype(jnp.float32)
    logits = drop * logits_ref[...] + (1.0 - drop) * preds_ref[...]
    m = jnp.max(logits, axis=1, keepdims=True)
    lse = jnp.log(jnp.sum(jnp.exp(logits - m), axis=1, keepdims=True)) + m
    out_ref[...] = -jnp.sum(oh_ref[...] * (logits - lse), axis=(0, 1),
                            keepdims=True) / float(n_rows)


@jax.jit
def kernel(features, preds, labels, oh_labels, Wc, bc):
    N, C, H, W = features.shape
    HW = H * W
    K = preds.shape[1]
    ft = jnp.transpose(features, (2, 3, 0, 1)).reshape(HW, N, C)

    Kp = ((K + 127) // 128) * 128
    oh_p = jnp.pad(oh_labels, ((0, 0), (0, Kp - K)))
    wc_p = jnp.pad(Wc, ((0, 0), (0, Kp - K)))
    G = pl.pallas_call(
        _g_body,
        out_shape=jax.ShapeDtypeStruct((N, C), jnp.float32),
    )(oh_p, wc_p)

    B = 32
    keep_cnt = float(math.ceil(HW / 3.0) + 1)
    pooled = pl.pallas_call(
        functools.partial(_pool_body, keep_cnt=keep_cnt, inv_hw=1.0 / HW),
        grid=(N // B,),
        in_specs=[
            pl.BlockSpec((HW, B, C), lambda i: (0, i, 0)),
            pl.BlockSpec((B, C), lambda i: (i, 0)),
        ],
        out_specs=pl.BlockSpec((B, C), lambda i: (i, 0)),
        out_shape=jax.ShapeDtypeStruct((N, C), jnp.float32),
    )(ft, G)

    logits_after, change = pl.pallas_call(
        functools.partial(_change_body, eps=1e-4),
        out_shape=(jax.ShapeDtypeStruct((N, K), jnp.float32),
                   jax.ShapeDtypeStruct((N, 1), jnp.float32)),
    )(pooled, Wc, bc.reshape(1, K), preds, oh_labels)

    th_idx = int(round(float(N) * 0.3333))
    loss = pl.pallas_call(
        functools.partial(_loss_body, n_rows=N, th_idx=th_idx),
        out_shape=jax.ShapeDtypeStruct((1, 1), jnp.float32),
    )(change, change.reshape(1, N), logits_after, preds, oh_labels)
    return loss[0, 0]


# speedup vs baseline: 7.4543x; 1.3162x over previous
import functools
import math

import jax
import jax.numpy as jnp
from jax.experimental import pallas as pl


def _g_body(oht_ref, wct_ref, g_ref):
    g_ref[...] = jax.lax.dot_general(
        oht_ref[...], wct_ref[...],
        dimension_numbers=(((0,), (0,)), ((), ())),
        preferred_element_type=jnp.float32)


def _pool_body(f_ref, g_ref, pooled_ref, *, keep_cnt, inv_hw):
    ft = f_ref[...]
    g = g_ref[...]
    sm = jnp.sum(ft * g[None, :, :], axis=2)
    smt = jnp.transpose(sm)
    hw = sm.shape[0]
    cnt = jnp.zeros_like(smt)
    for j in range(hw):
        col_j = jnp.broadcast_to(smt[:, j:j + 1], smt.shape)
        cnt += (col_j >= smt).astype(jnp.float32)
    mask = jnp.transpose((cnt >= keep_cnt).astype(jnp.float32))
    pooled_ref[...] = jnp.sum(ft * mask[:, :, None], axis=0) * inv_hw


def _softmax0(x):
    m = jnp.max(x, axis=0, keepdims=True)
    e = jnp.exp(x - m)
    return e / jnp.sum(e, axis=0, keepdims=True)


def _loss_body(pooled_ref, wct_ref, bct_ref, predst_ref, oht_ref, out_ref,
               *, n_rows, th_idx, eps):
    pooled = pooled_ref[...]
    logits_t = jax.lax.dot_general(
        wct_ref[...], pooled,
        dimension_numbers=(((1,), (1,)), ((), ())),
        preferred_element_type=jnp.float32) + bct_ref[...]
    preds_t = predst_ref[...]
    oh_t = oht_ref[...]
    before = jnp.sum(oh_t * _softmax0(preds_t), axis=0, keepdims=True)
    after = jnp.sum(oh_t * _softmax0(logits_t), axis=0, keepdims=True)
    change = jnp.maximum(before - after - eps, 0.0)
    change_col = jnp.transpose(change)
    cmp = (jnp.broadcast_to(change_col, (n_rows, n_rows)) <
           jnp.broadcast_to(change, (n_rows, n_rows)))
    cnt_lt = jnp.sum(cmp.astype(jnp.float32), axis=0, keepdims=True)
    drop = (cnt_lt >= float(n_rows - th_idx)).astype(jnp.float32)
    logits = drop * logits_t + (1.0 - drop) * preds_t
    m = jnp.max(logits, axis=0, keepdims=True)
    lse = jnp.log(jnp.sum(jnp.exp(logits - m), axis=0, keepdims=True)) + m
    out_ref[...] = -jnp.sum(oh_t * (logits - lse), axis=(0, 1),
                            keepdims=True) / float(n_rows)


@jax.jit
def kernel(features, preds, labels, oh_labels, Wc, bc):
    N, C, H, W = features.shape
    HW = H * W
    K = preds.shape[1]
    ft = jnp.transpose(features, (2, 3, 0, 1)).reshape(HW, N, C)
    oh_t = jnp.transpose(oh_labels)
    wc_t = jnp.transpose(Wc)
    preds_t = jnp.transpose(preds)

    G = pl.pallas_call(
        _g_body,
        out_shape=jax.ShapeDtypeStruct((N, C), jnp.float32),
    )(oh_t, wc_t)

    B = 64
    keep_cnt = float(math.ceil(HW / 3.0) + 1)
    pooled = pl.pallas_call(
        functools.partial(_pool_body, keep_cnt=keep_cnt, inv_hw=1.0 / HW),
        grid=(N // B,),
        in_specs=[
            pl.BlockSpec((HW, B, C), lambda i: (0, i, 0)),
            pl.BlockSpec((B, C), lambda i: (i, 0)),
        ],
        out_specs=pl.BlockSpec((B, C), lambda i: (i, 0)),
        out_shape=jax.ShapeDtypeStruct((N, C), jnp.float32),
    )(ft, G)

    th_idx = int(round(float(N) * 0.3333))
    loss = pl.pallas_call(
        functools.partial(_loss_body, n_rows=N, th_idx=th_idx, eps=1e-4),
        out_shape=jax.ShapeDtypeStruct((1, 1), jnp.float32),
    )(pooled, wc_t, bc.reshape(K, 1), preds_t, oh_t)
    return loss[0, 0]
